# baseline (speedup 1.0000x reference)
import functools

import jax
import jax.numpy as jnp
from jax import lax
from jax.experimental import pallas as pl
from jax.experimental.pallas import tpu as pltpu
from jax.experimental.pallas import tpu_sc as plsc

_NC = 2
_NS = 16
_NW = _NC * _NS
_L = 16


def _sc_embed_bag(xt, emb_table, w_scaled, b_pad, *, B, S, D):
    cols_per_w = B // _NW
    half = S // 2
    nk = D // _L
    mesh = plsc.VectorSubcoreMesh(core_axis_name="c", subcore_axis_name="s")

    @functools.partial(
        pl.kernel,
        out_type=jax.ShapeDtypeStruct((B, _L), jnp.float32),
        mesh=mesh,
        compiler_params=pltpu.CompilerParams(
            needs_layout_passes=False, use_tc_tiling_on_sc=False),
        scratch_types=[
            pltpu.VMEM((cols_per_w, 2, half), jnp.int32),
            pltpu.VMEM((S, D), jnp.float32),
            pltpu.VMEM((cols_per_w, _L), jnp.float32),
            pltpu.VMEM((2, D), jnp.float32),
            pltpu.VMEM((_L,), jnp.float32),
            pltpu.SemaphoreType.DMA,
        ],
    )
    def body(xt_hbm, table_hbm, w_hbm, b_hbm, out_hbm,
             idx_v, rows_v, out_v, w_v, b_v, sem):
        wid = lax.axis_index("s") * _NC + lax.axis_index("c")
        base = wid * cols_per_w
        pltpu.sync_copy(xt_hbm.at[pl.ds(base, cols_per_w)], idx_v)
        pltpu.sync_copy(w_hbm, w_v)
        pltpu.sync_copy(b_hbm, b_v)

        w0 = [w_v[0, pl.ds(k * _L, _L)] for k in range(nk)]
        w1 = [w_v[1, pl.ds(k * _L, _L)] for k in range(nk)]
        bvec = b_v[pl.ds(0, _L)]
        lanes = lax.iota(jnp.int32, _L)

        def col_body(c, carry):
            pltpu.async_copy(table_hbm.at[idx_v.at[c, 0]],
                             rows_v.at[pl.ds(0, half)], sem).wait()
            pltpu.async_copy(table_hbm.at[idx_v.at[c, 1]],
                             rows_v.at[pl.ds(half, half)], sem).wait()

            def s_body(s, accs):
                return tuple(a + rows_v[s, pl.ds(k * _L, _L)]
                             for k, a in enumerate(accs))

            z = jnp.zeros((_L,), jnp.float32)
            accs = lax.fori_loop(0, S, s_body, (z,) * nk, unroll=8)
            t0 = accs[0] * w0[0]
            t1 = accs[0] * w1[0]
            for k in range(1, nk):
                t0 = t0 + accs[k] * w0[k]
                t1 = t1 + accs[k] * w1[k]
            s0 = jnp.sum(t0)
            s1 = jnp.sum(t1)
            outvec = bvec + jnp.where(
                lanes == 0, s0, jnp.where(lanes == 1, s1, 0.0))
            out_v[c] = outvec
            return carry

        lax.fori_loop(0, cols_per_w, col_body, 0)
        pltpu.sync_copy(out_v, out_hbm.at[pl.ds(base, cols_per_w)])

    return body(xt, emb_table, w_scaled, b_pad)


def kernel(x, emb_table, fc_w, fc_b):
    S, B = x.shape
    N, D = emb_table.shape
    O = fc_w.shape[0]
    xt = x.T.astype(jnp.int32).reshape(B, 2, S // 2)
    w_scaled = (fc_w.astype(jnp.float32) / jnp.float32(S))
    b_pad = jnp.zeros((_L,), jnp.float32).at[:O].set(fc_b.astype(jnp.float32))
    out16 = _sc_embed_bag(xt, emb_table, w_scaled, b_pad, B=B, S=S, D=D)
    return out16[:, :O]


# speedup vs baseline: 1.3027x; 1.3027x over previous
import functools

import jax
import jax.numpy as jnp
from jax import lax
from jax.experimental import pallas as pl
from jax.experimental.pallas import tpu as pltpu
from jax.experimental.pallas import tpu_sc as plsc

_NC = 2
_NS = 16
_NW = _NC * _NS
_L = 16


def _sc_embed_bag(xt, emb_table, w_scaled, b_pad, *, B, S, D):
    cols_per_w = B // _NW
    half = S // 2
    nk = D // _L
    mesh = plsc.VectorSubcoreMesh(core_axis_name="c", subcore_axis_name="s")

    G = 1
    NBUF = 4
    ngroups = cols_per_w // G

    @functools.partial(
        pl.kernel,
        out_type=jax.ShapeDtypeStruct((B, _L), jnp.float32),
        mesh=mesh,
        compiler_params=pltpu.CompilerParams(
            needs_layout_passes=False, use_tc_tiling_on_sc=False),
        scratch_types=[
            pltpu.VMEM((cols_per_w, 2, half), jnp.int32),
            pltpu.VMEM((NBUF, G * S, D), jnp.float32),
            pltpu.VMEM((cols_per_w, _L), jnp.float32),
            pltpu.VMEM((2, D), jnp.float32),
            pltpu.VMEM((_L,), jnp.float32),
            pltpu.SemaphoreType.DMA((NBUF,)),
        ],
    )
    def body(xt_hbm, table_hbm, w_hbm, b_hbm, out_hbm,
             idx_v, rows_v, out_v, w_v, b_v, sem):
        wid = lax.axis_index("s") * _NC + lax.axis_index("c")
        base = wid * cols_per_w
        pltpu.sync_copy(xt_hbm.at[pl.ds(base, cols_per_w)], idx_v)
        pltpu.sync_copy(w_hbm, w_v)
        pltpu.sync_copy(b_hbm, b_v)

        w0 = [w_v[0, pl.ds(k * _L, _L)] for k in range(nk)]
        w1 = [w_v[1, pl.ds(k * _L, _L)] for k in range(nk)]
        bvec = b_v[pl.ds(0, _L)]
        lanes = lax.iota(jnp.int32, _L)

        def _chunks(g, b):
            for j in range(2 * G):
                src = table_hbm.at[idx_v.at[g * G + j // 2, j % 2]]
                dst = rows_v.at[b].at[pl.ds((j // 2) * S + (j % 2) * half, half)]
                yield src, dst

        def issue(g, b):
            for src, dst in _chunks(g, b):
                pltpu.async_copy(src, dst, sem.at[b])

        def drain(g, b):
            for src, dst in _chunks(g, b):
                pltpu.make_async_copy(src, dst, sem.at[b]).wait()

        for b in range(min(NBUF, ngroups)):
            issue(b, b)

        def group_body(gp, carry):
            for b in range(NBUF):
                g = gp * NBUF + b
                drain(g, b)
                for jc in range(G):
                    c = g * G + jc

                    def s_body(s, accs):
                        return tuple(
                            a + rows_v[b, jc * S + s, pl.ds(k * _L, _L)]
                            for k, a in enumerate(accs))

                    z = jnp.zeros((_L,), jnp.float32)
                    accs = lax.fori_loop(0, S, s_body, (z,) * nk, unroll=8)
                    t0 = accs[0] * w0[0]
                    t1 = accs[0] * w1[0]
                    for k in range(1, nk):
                        t0 = t0 + accs[k] * w0[k]
                        t1 = t1 + accs[k] * w1[k]
                    s0 = jnp.sum(t0)
                    s1 = jnp.sum(t1)
                    out_v[c] = bvec + jnp.where(
                        lanes == 0, s0, jnp.where(lanes == 1, s1, 0.0))

                @pl.when(g < ngroups - NBUF)
                def _():
                    issue(g + NBUF, b)
            return carry

        lax.fori_loop(0, ngroups // NBUF, group_body, 0)
        pltpu.sync_copy(out_v, out_hbm.at[pl.ds(base, cols_per_w)])

    return body(xt, emb_table, w_scaled, b_pad)


def kernel(x, emb_table, fc_w, fc_b):
    S, B = x.shape
    N, D = emb_table.shape
    O = fc_w.shape[0]
    xt = x.T.astype(jnp.int32).reshape(B, 2, S // 2)
    w_scaled = (fc_w.astype(jnp.float32) / jnp.float32(S))
    b_pad = jnp.zeros((_L,), jnp.float32).at[:O].set(fc_b.astype(jnp.float32))
    out16 = _sc_embed_bag(xt, emb_table, w_scaled, b_pad, B=B, S=S, D=D)
    return out16[:, :O]


# speedup vs baseline: 1.3029x; 1.0001x over previous
import functools

import jax
import jax.numpy as jnp
from jax import lax
from jax.experimental import pallas as pl
from jax.experimental.pallas import tpu as pltpu
from jax.experimental.pallas import tpu_sc as plsc

_NC = 2
_NS = 16
_NW = _NC * _NS
_L = 16


def _sc_embed_bag(x, emb_table, w_scaled, b_pad, *, B, S, D):
    cols_per_w = B // _NW
    nk = D // _L
    NBUF = 4
    mesh = plsc.VectorSubcoreMesh(core_axis_name="c", subcore_axis_name="s")

    @functools.partial(
        pl.kernel,
        out_type=jax.ShapeDtypeStruct((B, _L), jnp.float32),
        mesh=mesh,
        compiler_params=pltpu.CompilerParams(
            needs_layout_passes=False, use_tc_tiling_on_sc=False),
        scratch_types=[
            pltpu.VMEM((S, cols_per_w), jnp.int32),
            pltpu.VMEM((cols_per_w * S,), jnp.int32),
            pltpu.VMEM((NBUF, S, D), jnp.float32),
            pltpu.VMEM((cols_per_w, _L), jnp.float32),
            pltpu.VMEM((2, D), jnp.float32),
            pltpu.VMEM((_L,), jnp.float32),
            pltpu.SemaphoreType.DMA((NBUF,)),
        ],
    )
    def body(x_hbm, table_hbm, w_hbm, b_hbm, out_hbm,
             xblk_v, idx_v, rows_v, out_v, w_v, b_v, sem):
        wid = lax.axis_index("s") * _NC + lax.axis_index("c")
        base = wid * cols_per_w
        pltpu.sync_copy(x_hbm.at[:, pl.ds(base, cols_per_w)], xblk_v)
        pltpu.sync_copy(w_hbm, w_v)
        pltpu.sync_copy(b_hbm, b_v)

        lanes = lax.iota(jnp.int32, _L)
        col_off = [(h * _L + lanes) * S for h in range(cols_per_w // _L)]

        def t_body(s, carry):
            for h in range(cols_per_w // _L):
                val = xblk_v[s, pl.ds(h * _L, _L)]
                plsc.store_scatter(idx_v, [col_off[h] + s], val)
            return carry

        lax.fori_loop(0, S, t_body, 0)

        w0 = [w_v[0, pl.ds(k * _L, _L)] for k in range(nk)]
        w1 = [w_v[1, pl.ds(k * _L, _L)] for k in range(nk)]
        bvec = b_v[pl.ds(0, _L)]

        splits = [(0, 104), (104, S - 104)] if S > 128 else [(0, S)]

        def _chunks(c, b):
            for off, n in splits:
                src = table_hbm.at[idx_v.at[pl.ds(c * S + off, n)]]
                dst = rows_v.at[b].at[pl.ds(off, n)]
                yield src, dst

        def issue(c, b):
            for src, dst in _chunks(c, b):
                pltpu.async_copy(src, dst, sem.at[b])

        def drain(c, b):
            for src, dst in _chunks(c, b):
                pltpu.make_async_copy(src, dst, sem.at[b]).wait()

        for b in range(NBUF):
            issue(b, b)

        def group_body(gp, carry):
            for b in range(NBUF):
                c = gp * NBUF + b
                drain(c, b)

                def s_body(s, accs):
                    return tuple(a + rows_v[b, s, pl.ds(k * _L, _L)]
                                 for k, a in enumerate(accs))

                z = jnp.zeros((_L,), jnp.float32)
                accs = lax.fori_loop(0, S, s_body, (z,) * nk, unroll=8)
                t0 = accs[0] * w0[0]
                t1 = accs[0] * w1[0]
                for k in range(1, nk):
                    t0 = t0 + accs[k] * w0[k]
                    t1 = t1 + accs[k] * w1[k]
                s0 = jnp.sum(t0)
                s1 = jnp.sum(t1)
                out_v[c] = bvec + jnp.where(
                    lanes == 0, s0, jnp.where(lanes == 1, s1, 0.0))

                @pl.when(c < cols_per_w - NBUF)
                def _():
                    issue(c + NBUF, b)
            return carry

        lax.fori_loop(0, cols_per_w // NBUF, group_body, 0)
        pltpu.sync_copy(out_v, out_hbm.at[pl.ds(base, cols_per_w)])

    return body(x, emb_table, w_scaled, b_pad)


def kernel(x, emb_table, fc_w, fc_b):
    S, B = x.shape
    N, D = emb_table.shape
    O = fc_w.shape[0]
    w_scaled = (fc_w.astype(jnp.float32) / jnp.float32(S))
    b_pad = jnp.zeros((_L,), jnp.float32).at[:O].set(fc_b.astype(jnp.float32))
    out16 = _sc_embed_bag(x.astype(jnp.int32), emb_table, w_scaled, b_pad,
                          B=B, S=S, D=D)
    return out16[:, :O]


# speedup vs baseline: 1.3036x; 1.0006x over previous
import functools

import jax
import jax.numpy as jnp
from jax import lax
from jax.experimental import pallas as pl
from jax.experimental.pallas import tpu as pltpu
from jax.experimental.pallas import tpu_sc as plsc

_NC = 2
_NS = 16
_NW = _NC * _NS
_L = 16


def _sc_embed_bag(xt, emb_table, w_scaled, b_pad, *, B, S, D):
    cols_per_w = B // _NW
    nk = D // _L
    NBUF = 6
    mesh = plsc.VectorSubcoreMesh(core_axis_name="c", subcore_axis_name="s")

    @functools.partial(
        pl.kernel,
        out_type=jax.ShapeDtypeStruct((B, _L), jnp.float32),
        mesh=mesh,
        compiler_params=pltpu.CompilerParams(
            needs_layout_passes=False, use_tc_tiling_on_sc=False),
        scratch_types=[
            pltpu.VMEM((cols_per_w * S,), jnp.int32),
            pltpu.VMEM((NBUF, S, D), jnp.float32),
            pltpu.VMEM((cols_per_w, _L), jnp.float32),
            pltpu.VMEM((2, D), jnp.float32),
            pltpu.VMEM((_L,), jnp.float32),
            pltpu.SemaphoreType.DMA((NBUF,)),
        ],
    )
    def body(xt_hbm, table_hbm, w_hbm, b_hbm, out_hbm,
             idx_v, rows_v, out_v, w_v, b_v, sem):
        wid = lax.axis_index("s") * _NC + lax.axis_index("c")
        base = wid * cols_per_w
        pltpu.sync_copy(xt_hbm.at[pl.ds(base * S, cols_per_w * S)], idx_v)
        pltpu.sync_copy(w_hbm, w_v)
        pltpu.sync_copy(b_hbm, b_v)

        w0 = [w_v[0, pl.ds(k * _L, _L)] for k in range(nk)]
        w1 = [w_v[1, pl.ds(k * _L, _L)] for k in range(nk)]
        bvec = b_v[pl.ds(0, _L)]
        lanes = lax.iota(jnp.int32, _L)

        def _pair(c, b):
            src = table_hbm.at[idx_v.at[pl.ds(c * S, S)]]
            dst = rows_v.at[b]
            return src, dst

        def issue(c, b):
            src, dst = _pair(c, b)
            pltpu.async_copy(src, dst, sem.at[b])

        def drain(c, b):
            src, dst = _pair(c, b)
            pltpu.make_async_copy(src, dst, sem.at[b]).wait()

        for b in range(NBUF):
            issue(b, b)

        def group_body(gp, carry):
            for b in range(NBUF):
                c = gp * NBUF + b
                drain(c, b)

                def s_body(s, accs):
                    return tuple(a + rows_v[b, s, pl.ds(k * _L, _L)]
                                 for k, a in enumerate(accs))

                z = jnp.zeros((_L,), jnp.float32)
                accs = lax.fori_loop(0, S, s_body, (z,) * nk, unroll=8)
                t0 = accs[0] * w0[0]
                t1 = accs[0] * w1[0]
                for k in range(1, nk):
                    t0 = t0 + accs[k] * w0[k]
                    t1 = t1 + accs[k] * w1[k]
                s0 = jnp.sum(t0)
                s1 = jnp.sum(t1)
                out_v[c] = bvec + jnp.where(
                    lanes == 0, s0, jnp.where(lanes == 1, s1, 0.0))

                @pl.when(c < cols_per_w - NBUF)
                def _():
                    issue(c + NBUF, b)
            return carry

        main = (cols_per_w // NBUF) * NBUF
        lax.fori_loop(0, cols_per_w // NBUF, group_body, 0)
        for c in range(main, cols_per_w):
            b = c - main
            drain(c, b % NBUF)

            def s_body2(s, accs):
                return tuple(a + rows_v[b % NBUF, s, pl.ds(k * _L, _L)]
                             for k, a in enumerate(accs))

            z = jnp.zeros((_L,), jnp.float32)
            accs = lax.fori_loop(0, S, s_body2, (z,) * nk, unroll=8)
            t0 = accs[0] * w0[0]
            t1 = accs[0] * w1[0]
            for k in range(1, nk):
                t0 = t0 + accs[k] * w0[k]
                t1 = t1 + accs[k] * w1[k]
            s0 = jnp.sum(t0)
            s1 = jnp.sum(t1)
            out_v[c] = bvec + jnp.where(
                lanes == 0, s0, jnp.where(lanes == 1, s1, 0.0))

        pltpu.sync_copy(out_v, out_hbm.at[pl.ds(base, cols_per_w)])

    return body(xt, emb_table, w_scaled, b_pad)


def kernel(x, emb_table, fc_w, fc_b):
    S, B = x.shape
    N, D = emb_table.shape
    O = fc_w.shape[0]
    xt = x.T.astype(jnp.int32).reshape(B * S)
    w_scaled = (fc_w.astype(jnp.float32) / jnp.float32(S))
    b_pad = jnp.zeros((_L,), jnp.float32).at[:O].set(fc_b.astype(jnp.float32))
    out16 = _sc_embed_bag(xt, emb_table, w_scaled, b_pad, B=B, S=S, D=D)
    return out16[:, :O]


# speedup vs baseline: 1.3791x; 1.0579x over previous
import functools

import jax
import jax.numpy as jnp
from jax import lax
from jax.experimental import pallas as pl
from jax.experimental.pallas import tpu as pltpu
from jax.experimental.pallas import tpu_sc as plsc

_NC = 2
_NS = 16
_NW = _NC * _NS
_L = 16
_CHUNK = 1024


def _project(tT, fc_w, *, N, D):
    nchunks = (N + _CHUNK - 1) // _CHUNK

    def body(w_ref, t_ref, out_ref):
        prod = jnp.dot(w_ref[...], t_ref[...],
                       preferred_element_type=jnp.float32)
        out_ref[...] = prod.reshape(1, 2, _CHUNK // 128, 128)

    return pl.pallas_call(
        body,
        grid=(nchunks,),
        in_specs=[
            pl.BlockSpec((2, D), lambda g: (0, 0)),
            pl.BlockSpec((D, _CHUNK), lambda g: (0, g)),
        ],
        out_specs=pl.BlockSpec((1, 2, _CHUNK // 128, 128),
                               lambda g: (g, 0, 0, 0)),
        out_shape=jax.ShapeDtypeStruct(
            (nchunks, 2, _CHUNK // 128, 128), jnp.float32),
    )(fc_w, tT)


def _sc_lookup(xt, t2, b_pad, *, B, S, N, nchunks):
    cols_per_w = B // _NW
    SPAD = 208
    NBUF = 8
    chunks_per_tec = nchunks // _NS
    tail_chunks = nchunks - chunks_per_tec * _NS
    mesh = plsc.VectorSubcoreMesh(core_axis_name="c", subcore_axis_name="s")

    @functools.partial(
        pl.kernel,
        out_type=jax.ShapeDtypeStruct((B, _L), jnp.float32),
        mesh=mesh,
        compiler_params=pltpu.CompilerParams(
            needs_layout_passes=False, use_tc_tiling_on_sc=False),
        scratch_types=[
            pltpu.VMEM((cols_per_w * S,), jnp.int32),
            pltpu.VMEM((2, _CHUNK // 128, 128), jnp.float32),
            pltpu.VMEM((_CHUNK,), jnp.uint32),
            pltpu.VMEM((NBUF, SPAD), jnp.uint32),
            pltpu.VMEM((cols_per_w, _L), jnp.float32),
            pltpu.VMEM((_L,), jnp.float32),
            pltpu.VMEM_SHARED((nchunks * _CHUNK,), jnp.uint32),
            pltpu.SemaphoreType.DMA((NBUF,)),
        ],
    )
    def body(xt_hbm, t2_hbm, b_hbm, out_hbm,
             idx_v, chunk_v, pk_v, rows_v, out_v, b_v, pt_sp, sem):
        cid = lax.axis_index("c")
        sid = lax.axis_index("s")
        wid = sid * _NC + cid
        base = wid * cols_per_w
        pltpu.sync_copy(xt_hbm.at[pl.ds(base * S, cols_per_w * S)], idx_v)
        pltpu.sync_copy(b_hbm, b_v)

        def pack_chunk(c, carry):
            pltpu.sync_copy(t2_hbm.at[c], chunk_v)
            for k in range(_CHUNK // _L):
                s8, l0 = divmod(k * _L, 128)
                a = chunk_v[0, s8, pl.ds(l0, _L)]
                b = chunk_v[1, s8, pl.ds(l0, _L)]
                au = plsc.bitcast(a, jnp.uint32)
                bu = plsc.bitcast(b, jnp.uint32)
                half = jnp.uint32(0x8000)
                top = jnp.uint32(0xFFFF0000)
                word = ((au + half) >> 16) | ((bu + half) & top)
                pk_v[pl.ds(k * _L, _L)] = word
            pltpu.sync_copy(pk_v, pt_sp.at[pl.ds(c * _CHUNK, _CHUNK)])
            return carry

        lax.fori_loop(sid * chunks_per_tec, (sid + 1) * chunks_per_tec,
                      pack_chunk, 0)
        for t in range(tail_chunks):
            @pl.when(sid == _NS - 1)
            def _():
                pack_chunk(_NS * chunks_per_tec + t, 0)
        plsc.subcore_barrier()

        bvec = b_v[pl.ds(0, _L)]
        lanes = lax.iota(jnp.int32, _L)
        inv_s = jnp.float32(1.0 / S)

        def _pair(c, b):
            src = pt_sp.at[idx_v.at[pl.ds(c * S, S)]]
            dst = rows_v.at[b].at[pl.ds(0, S)]
            return src, dst

        def issue(c, b):
            src, dst = _pair(c, b)
            pltpu.async_copy(src, dst, sem.at[b])

        def drain(c, b):
            src, dst = _pair(c, b)
            pltpu.make_async_copy(src, dst, sem.at[b]).wait()

        zero32 = jnp.zeros((_L,), jnp.uint32)
        for b in range(NBUF):
            rows_v[b, pl.ds(SPAD - _L, _L)] = zero32
            issue(b, b)

        def col_accum(c, b):
            za = jnp.zeros((_L,), jnp.float32)
            zb = jnp.zeros((_L,), jnp.float32)

            def s_body(k, accs):
                aa, ab = accs
                w = rows_v[b, pl.ds(k * _L, _L)]
                av = plsc.bitcast(w << 16, jnp.float32)
                bv = plsc.bitcast(w & jnp.uint32(0xFFFF0000), jnp.float32)
                return (aa + av, ab + bv)

            aa, ab = lax.fori_loop(0, SPAD // _L, s_body, (za, zb), unroll=13)
            s0 = jnp.sum(aa) * inv_s
            s1 = jnp.sum(ab) * inv_s
            out_v[c] = bvec + jnp.where(
                lanes == 0, s0, jnp.where(lanes == 1, s1, 0.0))

        def group_body(gp, carry):
            for b in range(NBUF):
                c = gp * NBUF + b
                drain(c, b)
                col_accum(c, b)

                @pl.when(c < cols_per_w - NBUF)
                def _():
                    issue(c + NBUF, b)
            return carry

        lax.fori_loop(0, cols_per_w // NBUF, group_body, 0)
        pltpu.sync_copy(out_v, out_hbm.at[pl.ds(base, cols_per_w)])

    return body(xt, t2, b_pad)


def kernel(x, emb_table, fc_w, fc_b):
    S, B = x.shape
    N, D = emb_table.shape
    O = fc_w.shape[0]
    t2 = _project(emb_table.T, fc_w.astype(jnp.float32), N=N, D=D)
    nchunks = t2.shape[0]
    xt = x.T.astype(jnp.int32).reshape(B * S)
    b_pad = jnp.zeros((_L,), jnp.float32).at[:O].set(fc_b.astype(jnp.float32))
    out16 = _sc_lookup(xt, t2, b_pad, B=B, S=S, N=N, nchunks=nchunks)
    return out16[:, :O]


# speedup vs baseline: 5.2195x; 3.7847x over previous
import functools

import jax
import jax.numpy as jnp
from jax import lax
from jax.experimental import pallas as pl
from jax.experimental.pallas import tpu as pltpu
from jax.experimental.pallas import tpu_sc as plsc

_NC = 2
_NS = 16
_NW = _NC * _NS
_L = 16
_CHUNK = 16384
_SUB = 4096


def _project(tT, fc_w, *, N, D):
    nchunks = (N + _CHUNK - 1) // _CHUNK

    def body(w_ref, t_ref, out_ref):
        prod = jnp.dot(w_ref[...], t_ref[...],
                       preferred_element_type=jnp.float32)
        out_ref[...] = prod.reshape(1, 2, _CHUNK // 128, 128)

    return pl.pallas_call(
        body,
        grid=(nchunks,),
        in_specs=[
            pl.BlockSpec((2, D), lambda g: (0, 0)),
            pl.BlockSpec((D, _CHUNK), lambda g: (0, g)),
        ],
        out_specs=pl.BlockSpec((1, 2, _CHUNK // 128, 128),
                               lambda g: (g, 0, 0, 0)),
        out_shape=jax.ShapeDtypeStruct(
            (nchunks, 2, _CHUNK // 128, 128), jnp.float32),
    )(fc_w, tT)


def _sc_lookup(xt, t2, b_pad, *, B, S, N, nchunks):
    cols_per_w = B // _NW
    SPAD = 208
    NBUF = 8
    rounds = (nchunks + _NS - 1) // _NS
    mesh = plsc.VectorSubcoreMesh(core_axis_name="c", subcore_axis_name="s")

    @functools.partial(
        pl.kernel,
        out_type=jax.ShapeDtypeStruct((B, _L), jnp.float32),
        mesh=mesh,
        compiler_params=pltpu.CompilerParams(
            needs_layout_passes=False, use_tc_tiling_on_sc=False),
        scratch_types=[
            pltpu.VMEM((cols_per_w * S,), jnp.int32),
            pltpu.VMEM((2, _SUB // 128, 128), jnp.float32),
            pltpu.VMEM((_SUB,), jnp.uint32),
            pltpu.VMEM((NBUF, SPAD), jnp.uint32),
            pltpu.VMEM((cols_per_w, _L), jnp.float32),
            pltpu.VMEM((_L,), jnp.float32),
            pltpu.VMEM_SHARED((nchunks * _CHUNK,), jnp.uint32),
            pltpu.SemaphoreType.DMA((NBUF,)),
        ],
    )
    def body(xt_hbm, t2_hbm, b_hbm, out_hbm,
             idx_v, chunk_v, pk_v, rows_v, out_v, b_v, pt_sp, sem):
        cid = lax.axis_index("c")
        sid = lax.axis_index("s")
        wid = sid * _NC + cid
        base = wid * cols_per_w
        pltpu.sync_copy(xt_hbm.at[pl.ds(base * S, cols_per_w * S)], idx_v)
        pltpu.sync_copy(b_hbm, b_v)

        def pack_chunk(c):
            for u in range(_CHUNK // _SUB):
                pltpu.sync_copy(
                    t2_hbm.at[c].at[:, pl.ds(u * (_SUB // 128), _SUB // 128)],
                    chunk_v)

                def pack_body(k, carry):
                    s8 = k // 8
                    l0 = (k % 8) * _L
                    a = chunk_v[0, s8, pl.ds(l0, _L)]
                    b = chunk_v[1, s8, pl.ds(l0, _L)]
                    au = plsc.bitcast(a, jnp.uint32)
                    bu = plsc.bitcast(b, jnp.uint32)
                    half = jnp.uint32(0x8000)
                    top = jnp.uint32(0xFFFF0000)
                    word = ((au + half) >> 16) | ((bu + half) & top)
                    pk_v[pl.ds(k * _L, _L)] = word
                    return carry

                lax.fori_loop(0, _SUB // _L, pack_body, 0, unroll=8)
                pltpu.sync_copy(
                    pk_v, pt_sp.at[pl.ds(c * _CHUNK + u * _SUB, _SUB)])

        for t in range(rounds):
            c = t * _NS + sid

            @pl.when(c < nchunks)
            def _():
                pack_chunk(c)
        plsc.subcore_barrier()

        bvec = b_v[pl.ds(0, _L)]
        lanes = lax.iota(jnp.int32, _L)
        inv_s = jnp.float32(1.0 / S)

        def _pair(c, b):
            src = pt_sp.at[idx_v.at[pl.ds(c * S, S)]]
            dst = rows_v.at[b].at[pl.ds(0, S)]
            return src, dst

        def issue(c, b):
            src, dst = _pair(c, b)
            pltpu.async_copy(src, dst, sem.at[b])

        def drain(c, b):
            src, dst = _pair(c, b)
            pltpu.make_async_copy(src, dst, sem.at[b]).wait()

        zero32 = jnp.zeros((_L,), jnp.uint32)
        for b in range(NBUF):
            rows_v[b, pl.ds(SPAD - _L, _L)] = zero32
            issue(b, b)

        def col_accum(c, b):
            za = jnp.zeros((_L,), jnp.float32)
            zb = jnp.zeros((_L,), jnp.float32)

            def s_body(k, accs):
                aa, ab = accs
                w = rows_v[b, pl.ds(k * _L, _L)]
                av = plsc.bitcast(w << 16, jnp.float32)
                bv = plsc.bitcast(w & jnp.uint32(0xFFFF0000), jnp.float32)
                return (aa + av, ab + bv)

            aa, ab = lax.fori_loop(0, SPAD // _L, s_body, (za, zb), unroll=13)
            s0 = jnp.sum(aa) * inv_s
            s1 = jnp.sum(ab) * inv_s
            out_v[c] = bvec + jnp.where(
                lanes == 0, s0, jnp.where(lanes == 1, s1, 0.0))

        def group_body(gp, carry):
            for b in range(NBUF):
                c = gp * NBUF + b
                drain(c, b)
                col_accum(c, b)

                @pl.when(c < cols_per_w - NBUF)
                def _():
                    issue(c + NBUF, b)
            return carry

        lax.fori_loop(0, cols_per_w // NBUF, group_body, 0)
        pltpu.sync_copy(out_v, out_hbm.at[pl.ds(base, cols_per_w)])

    return body(xt, t2, b_pad)


def kernel(x, emb_table, fc_w, fc_b):
    S, B = x.shape
    N, D = emb_table.shape
    O = fc_w.shape[0]
    t2 = _project(emb_table.T, fc_w.astype(jnp.float32), N=N, D=D)
    nchunks = t2.shape[0]
    xt = x.T.astype(jnp.int32).reshape(B * S)
    b_pad = jnp.zeros((_L,), jnp.float32).at[:O].set(fc_b.astype(jnp.float32))
    out16 = _sc_lookup(xt, t2, b_pad, B=B, S=S, N=N, nchunks=nchunks)
    return out16[:, :O]


# speedup vs baseline: 6.0922x; 1.1672x over previous
import functools

import jax
import jax.numpy as jnp
from jax import lax
from jax.experimental import pallas as pl
from jax.experimental.pallas import tpu as pltpu
from jax.experimental.pallas import tpu_sc as plsc

_NC = 2
_NS = 16
_NW = _NC * _NS
_L = 16
_CHUNK = 32768
_SUB = 4096


def _project(tT, fc_w, *, N, D):
    nchunks = (N + _CHUNK - 1) // _CHUNK

    def body(w_ref, t_ref, out_ref):
        prod = jnp.dot(w_ref[...], t_ref[...],
                       preferred_element_type=jnp.float32)
        out_ref[...] = prod.reshape(1, 2, _CHUNK // 128, 128)

    return pl.pallas_call(
        body,
        grid=(nchunks,),
        in_specs=[
            pl.BlockSpec((2, D), lambda g: (0, 0)),
            pl.BlockSpec((D, _CHUNK), lambda g: (0, g)),
        ],
        out_specs=pl.BlockSpec((1, 2, _CHUNK // 128, 128),
                               lambda g: (g, 0, 0, 0)),
        out_shape=jax.ShapeDtypeStruct(
            (nchunks, 2, _CHUNK // 128, 128), jnp.float32),
    )(fc_w, tT)


def _sc_lookup(xt, t2, b_pad, *, B, S, N, nchunks):
    cols_per_w = B // _NW
    SPAD = 208
    NBUF = 8
    rounds = (nchunks + _NS - 1) // _NS
    mesh = plsc.VectorSubcoreMesh(core_axis_name="c", subcore_axis_name="s")

    @functools.partial(
        pl.kernel,
        out_type=jax.ShapeDtypeStruct((B, _L), jnp.float32),
        mesh=mesh,
        compiler_params=pltpu.CompilerParams(
            needs_layout_passes=False, use_tc_tiling_on_sc=False),
        scratch_types=[
            pltpu.VMEM((cols_per_w * S,), jnp.int32),
            pltpu.VMEM((2, 2, _SUB // 128, 128), jnp.float32),
            pltpu.VMEM((_SUB,), jnp.uint32),
            pltpu.VMEM((NBUF, SPAD), jnp.uint32),
            pltpu.VMEM((cols_per_w, _L), jnp.float32),
            pltpu.VMEM((_L,), jnp.float32),
            pltpu.VMEM_SHARED((nchunks * _CHUNK,), jnp.uint32),
            pltpu.SemaphoreType.DMA((NBUF,)),
            pltpu.SemaphoreType.DMA((2,)),
            pltpu.SemaphoreType.DMA,
        ],
    )
    def body(xt_hbm, t2_hbm, b_hbm, out_hbm,
             idx_v, chunk_v, pk_v, rows_v, out_v, b_v, pt_sp,
             sem, psem, isem):
        cid = lax.axis_index("c")
        sid = lax.axis_index("s")
        wid = sid * _NC + cid
        base = wid * cols_per_w
        idx_cp = pltpu.async_copy(
            xt_hbm.at[pl.ds(base * S, cols_per_w * S)], idx_v, isem)
        pltpu.sync_copy(b_hbm, b_v)

        nsub = _CHUNK // _SUB
        ntask = rounds * nsub

        def _sub_src(j, b):
            t, u = divmod(j, nsub)
            c = t * _NS + sid
            src = t2_hbm.at[c].at[:, pl.ds(u * (_SUB // 128), _SUB // 128)]
            return c, src, chunk_v.at[b]

        def _valid(j):
            t = j // nsub
            return (t * _NS + sid) < nchunks

        def issue_sub(j, b):
            @pl.when(_valid(j))
            def _():
                c, src, dst = _sub_src(j, b)
                pltpu.async_copy(src, dst, psem.at[b])

        issue_sub(0, 0)
        issue_sub(1, 1)
        for j in range(ntask):
            b = j % 2

            @pl.when(_valid(j))
            def _():
                c, src, dst = _sub_src(j, b)
                pltpu.make_async_copy(src, dst, psem.at[b]).wait()

                def pack_body(k, carry):
                    s8 = k // 8
                    l0 = (k % 8) * _L
                    a = chunk_v[b, 0, s8, pl.ds(l0, _L)]
                    bb = chunk_v[b, 1, s8, pl.ds(l0, _L)]
                    au = plsc.bitcast(a, jnp.uint32)
                    bu = plsc.bitcast(bb, jnp.uint32)
                    half = jnp.uint32(0x8000)
                    top = jnp.uint32(0xFFFF0000)
                    word = ((au + half) >> 16) | ((bu + half) & top)
                    pk_v[pl.ds(k * _L, _L)] = word
                    return carry

                lax.fori_loop(0, _SUB // _L, pack_body, 0, unroll=8)
                u = j % nsub
                pltpu.sync_copy(
                    pk_v, pt_sp.at[pl.ds(c * _CHUNK + u * _SUB, _SUB)])
            if j + 2 < ntask:
                issue_sub(j + 2, b)
        plsc.subcore_barrier()
        idx_cp.wait()

        bvec = b_v[pl.ds(0, _L)]
        lanes = lax.iota(jnp.int32, _L)
        inv_s = jnp.float32(1.0 / S)

        def _pair(c, b):
            src = pt_sp.at[idx_v.at[pl.ds(c * S, S)]]
            dst = rows_v.at[b].at[pl.ds(0, S)]
            return src, dst

        def issue(c, b):
            src, dst = _pair(c, b)
            pltpu.async_copy(src, dst, sem.at[b])

        def drain(c, b):
            src, dst = _pair(c, b)
            pltpu.make_async_copy(src, dst, sem.at[b]).wait()

        zero32 = jnp.zeros((_L,), jnp.uint32)
        for b in range(NBUF):
            rows_v[b, pl.ds(SPAD - _L, _L)] = zero32
            issue(b, b)

        def col_accum(c, b):
            za = jnp.zeros((_L,), jnp.float32)
            zb = jnp.zeros((_L,), jnp.float32)

            def s_body(k, accs):
                aa, ab = accs
                w = rows_v[b, pl.ds(k * _L, _L)]
                av = plsc.bitcast(w << 16, jnp.float32)
                bv = plsc.bitcast(w & jnp.uint32(0xFFFF0000), jnp.float32)
                return (aa + av, ab + bv)

            aa, ab = lax.fori_loop(0, SPAD // _L, s_body, (za, zb), unroll=13)
            s0 = jnp.sum(aa) * inv_s
            s1 = jnp.sum(ab) * inv_s
            out_v[c] = bvec + jnp.where(
                lanes == 0, s0, jnp.where(lanes == 1, s1, 0.0))

        def group_body(gp, carry):
            for b in range(NBUF):
                c = gp * NBUF + b
                drain(c, b)
                col_accum(c, b)

                @pl.when(c < cols_per_w - NBUF)
                def _():
                    issue(c + NBUF, b)
            return carry

        lax.fori_loop(0, cols_per_w // NBUF, group_body, 0)
        pltpu.sync_copy(out_v, out_hbm.at[pl.ds(base, cols_per_w)])

    return body(xt, t2, b_pad)


def kernel(x, emb_table, fc_w, fc_b):
    S, B = x.shape
    N, D = emb_table.shape
    O = fc_w.shape[0]
    t2 = _project(emb_table.T, fc_w.astype(jnp.float32), N=N, D=D)
    nchunks = t2.shape[0]
    xt = x.T.astype(jnp.int32).reshape(B * S)
    b_pad = jnp.zeros((_L,), jnp.float32).at[:O].set(fc_b.astype(jnp.float32))
    out16 = _sc_lookup(xt, t2, b_pad, B=B, S=S, N=N, nchunks=nchunks)
    return out16[:, :O]
